# baseline (speedup 1.0000x reference)
import functools

import jax
import jax.numpy as jnp
from jax import lax
from jax.experimental import pallas as pl
from jax.experimental.pallas import tpu as pltpu
from jax.experimental.pallas import tpu_sc as plsc

BATCH = 16384
D = 64
ITEMS = 1_000_000
NC = 2
NS = 16
NW = NC * NS
BPW = BATCH // NW
IPW = ITEMS // NW
NVREG = BATCH // 16
CAP = 16
NCOLP = 256
IRING = 4
ILOOK = 2
URING = 8
ULOOK = 4
TRASH = BATCH
SROWS = BATCH + 8
SCAN_BASE = 1

_mesh = plsc.VectorSubcoreMesh(core_axis_name="c", subcore_axis_name="s")


def _wid():
    return lax.axis_index("s") * NC + lax.axis_index("c")


def _splat(x):
    return jnp.broadcast_to(x, (16,))


@functools.partial(
    pl.kernel,
    mesh=_mesh,
    out_type=jax.ShapeDtypeStruct((SROWS, 128), jnp.float32),
    scratch_types=(
        [
            pltpu.VMEM((BATCH,), jnp.int32),
            pltpu.VMEM((NCOLP,), jnp.int32),
            pltpu.VMEM((NCOLP * CAP,), jnp.int32),
            pltpu.VMEM((NCOLP * CAP,), jnp.int32),
            pltpu.VMEM((BATCH,), jnp.int32),
            pltpu.VMEM((BATCH,), jnp.int32),
            pltpu.VMEM((IRING, D, 128), jnp.float32),
            pltpu.VMEM((128, 128), jnp.float32),
            pltpu.VMEM((128,), jnp.int32),
        ]
        + [pltpu.SemaphoreType.DMA] * (IRING + 1)
    ),
    compiler_params=pltpu.CompilerParams(needs_layout_passes=False,
                                         disable_bounds_checks=True),
)
def _gather_items(item_hbm, itemT_hbm, icols_hbm,
                  iidx_v, seen_v, ib_v, bb_v, oi_v, ob_v,
                  ring_v, stage_v, bpos_v, *sems):
    isems, ssem = sems[:IRING], sems[IRING]
    wid = _wid()
    lo = wid * IPW
    hi = lo + IPW
    col_lo = lo >> 7

    pltpu.sync_copy(item_hbm, iidx_v)

    lane = lax.iota(jnp.int32, 16)
    zero16 = jnp.zeros((16,), jnp.int32)
    for g in range(NCOLP // 16):
        seen_v[pl.ds(g * 16, 16)] = zero16

    def prefill_bpos():
        t16 = jnp.full((16,), TRASH, jnp.int32)
        for t in range(8):
            bpos_v[pl.ds(t * 16, 16)] = t16

    prefill_bpos()

    def scanb(k, ocnt):
        v = iidx_v[pl.ds(k * 16, 16)]
        m = (v >= lo) & (v < hi)
        c = jnp.where(m, (v >> 7) - col_lo, 0)
        cnt, lastm = plsc.scan_count(c, m)
        n = plsc.load_gather(seen_v, [c], mask=m)
        pos = n + cnt - SCAN_BASE
        posc = jnp.minimum(pos, CAP - 1)
        bvec = k * 16 + lane
        okm = m & (pos < CAP)
        slotv = c * CAP + posc
        plsc.store_scatter(ib_v, [slotv], v, mask=okm)
        plsc.store_scatter(bb_v, [slotv], bvec, mask=okm)
        om = m & (pos >= CAP)
        plsc.store_compressed(oi_v.at[pl.ds(ocnt, 16)], v, mask=om)
        plsc.store_compressed(ob_v.at[pl.ds(ocnt, 16)], bvec, mask=om)
        plsc.store_scatter(seen_v, [c], n + cnt + (1 - SCAN_BASE),
                           mask=m & lastm)
        return ocnt + plsc.all_reduce_population_count(om)[0]

    ocnt = lax.fori_loop(0, NVREG, scanb, 0)

    d16 = [lane + 16 * cb for cb in range(D // 16)]

    def fire_col(slot, ck, go):
        @pl.when(go)
        def _():
            off = pl.multiple_of((col_lo + ck) * 128, 128)
            pltpu.async_copy(itemT_hbm.at[:, pl.ds(off, 128)],
                             ring_v.at[slot], isems[slot])

    def flush():
        pltpu.async_copy(stage_v, icols_hbm.at[bpos_v], ssem).wait()
        prefill_bpos()

    def emit(slot, i_e, b_e, widx):
        l = _splat(i_e & 127)
        for cb in range(D // 16):
            stage_v[widx, pl.ds(cb * 16, 16)] = plsc.load_gather(
                ring_v.at[slot], [d16[cb], l])
        plsc.store_scatter(bpos_v, [_splat(widx)], _splat(b_e),
                           mask=lane == 0)
        widx = widx + 1

        @pl.when(widx == 128)
        def _():
            flush()

        return jnp.where(widx == 128, 0, widx)

    sv0 = seen_v[pl.ds(0, 16)]
    for p in range(ILOOK):
        fire_col(p % IRING, p, sv0[p] > 0)

    def colgroup(g, widx):
        p0 = g * 16
        sv = seen_v[pl.ds(p0, 16)]
        svn = seen_v[pl.ds(jnp.minimum(p0 + 16, NCOLP - 16), 16)]
        for k in range(16):
            pf = p0 + k + ILOOK
            cnt_f = sv[k + ILOOK] if k + ILOOK < 16 else svn[k + ILOOK - 16]
            fire_col((k + ILOOK) % IRING, pf, (pf < NCOLP) & (cnt_f > 0))

            slot = k % IRING
            cntk = sv[k]
            ckv = _splat(p0 + k)

            def body(widx, slot=slot, cntk=cntk, ckv=ckv):
                pltpu.make_async_copy(itemT_hbm.at[:, pl.ds(0, 128)],
                                      ring_v.at[slot], isems[slot]).wait()

                def elem(e, widx):
                    ev = ckv * CAP + _splat(e)
                    i_e = plsc.load_gather(ib_v, [ev])[0]
                    b_e = plsc.load_gather(bb_v, [ev])[0]
                    return emit(slot, i_e, b_e, widx)

                return lax.fori_loop(0, jnp.minimum(cntk, CAP), elem, widx)

            widx = lax.cond(cntk > 0, body, lambda w: w, widx)
        return widx

    widx = lax.fori_loop(0, NCOLP // 16, colgroup, 0)

    def oflow(e, widx):
        i_e = plsc.load_gather(oi_v, [_splat(e)])[0]
        b_e = plsc.load_gather(ob_v, [_splat(e)])[0]
        fire_col(0, (i_e >> 7) - col_lo, True)
        pltpu.make_async_copy(itemT_hbm.at[:, pl.ds(0, 128)],
                              ring_v.at[0], isems[0]).wait()
        return emit(0, i_e, b_e, widx)

    widx = lax.fori_loop(0, ocnt, oflow, widx)

    @pl.when(widx > 0)
    def _():
        flush()


@functools.partial(
    pl.kernel,
    mesh=_mesh,
    out_type=jax.ShapeDtypeStruct((BATCH,), jnp.float32),
    scratch_types=(
        [
            pltpu.VMEM((BPW,), jnp.int32),
            pltpu.VMEM((BPW, 128), jnp.float32),
            pltpu.VMEM((URING, 8, D), jnp.float32),
            pltpu.VMEM((BPW,), jnp.float32),
        ]
        + [pltpu.SemaphoreType.DMA] * (URING + 1)
    ),
    compiler_params=pltpu.CompilerParams(needs_layout_passes=False,
                                         disable_bounds_checks=True),
)
def _pair_dot(user_hbm, uemb_hbm, icols_hbm, out_hbm,
              uidx_v, vrows_v, uring_v, out_v, *sems):
    usems, vsem = sems[:URING], sems[URING]
    wid = _wid()
    base = wid * BPW

    pltpu.sync_copy(user_hbm.at[pl.ds(base, BPW)], uidx_v)
    vcopy = pltpu.async_copy(icols_hbm.at[pl.ds(base, BPW)], vrows_v, vsem)

    def fire(slot, uv):
        uoff = pl.multiple_of((uv >> 3) * 8, 8)
        pltpu.async_copy(uemb_hbm.at[pl.ds(uoff, 8), :],
                         uring_v.at[slot], usems[slot])

    uvec0 = uidx_v[pl.ds(0, 16)]
    for j in range(ULOOK):
        fire(j % URING, uvec0[j])
    vcopy.wait()

    lane = lax.iota(jnp.int32, 16)

    def block(blk, carry):
        j0 = blk * 16
        uvec = uidx_v[pl.ds(j0, 16)]
        uvec_n = uidx_v[pl.ds(jnp.minimum(j0 + 16, BPW - 16), 16)]
        acc = jnp.zeros((16,), jnp.float32)
        for r in range(16):
            j = j0 + r
            uvf = uvec[r + ULOOK] if r + ULOOK < 16 else uvec_n[r + ULOOK - 16]
            slot_f = (r + ULOOK) % URING

            @pl.when(j + ULOOK < BPW)
            def _():
                fire(slot_f, uvf)

            slot = r % URING
            pltpu.make_async_copy(uemb_hbm.at[pl.ds(0, 8), :],
                                  uring_v.at[slot], usems[slot]).wait()
            urow = uvec[r] & 7
            p = (uring_v[slot, urow, pl.ds(0, 16)]
                 * vrows_v[j, pl.ds(0, 16)])
            for cb in range(1, D // 16):
                p = p + (uring_v[slot, urow, pl.ds(cb * 16, 16)]
                         * vrows_v[j, pl.ds(cb * 16, 16)])
            acc = jnp.where(lane == r, jnp.sum(p), acc)
        out_v[pl.ds(j0, 16)] = acc
        return carry

    lax.fori_loop(0, BPW // 16, block, 0)
    pltpu.sync_copy(out_v, out_hbm.at[pl.ds(base, BPW)])


def kernel(user, item, user_emb, item_emb):
    icols = _gather_items(item.astype(jnp.int32), item_emb.T)
    return _pair_dot(user.astype(jnp.int32), user_emb, icols)


# speedup vs baseline: 1.1519x; 1.1519x over previous
import functools

import jax
import jax.numpy as jnp
from jax import lax
from jax.experimental import pallas as pl
from jax.experimental.pallas import tpu as pltpu
from jax.experimental.pallas import tpu_sc as plsc

BATCH = 16384
D = 64
ITEMS = 1_000_000
NC = 2
NS = 16
NW = NC * NS
BPW = BATCH // NW
IPW = ITEMS // NW
NVREG = BATCH // 16
CAP = 16
NCOLP = 256
IRING = 4
ILOOK = 3
URING = 8
ULOOK = 4
TRASH = BATCH
SROWS = BATCH + 8
SCAN_BASE = 1

_mesh = plsc.VectorSubcoreMesh(core_axis_name="c", subcore_axis_name="s")


def _wid():
    return lax.axis_index("s") * NC + lax.axis_index("c")


def _splat(x):
    return jnp.broadcast_to(x, (16,))


@functools.partial(
    pl.kernel,
    mesh=_mesh,
    out_type=jax.ShapeDtypeStruct((SROWS, 128), jnp.float32),
    scratch_types=(
        [
            pltpu.VMEM((BATCH,), jnp.int32),
            pltpu.VMEM((NCOLP,), jnp.int32),
            pltpu.VMEM((NCOLP * CAP,), jnp.int32),
            pltpu.VMEM((NCOLP * CAP,), jnp.int32),
            pltpu.VMEM((BATCH,), jnp.int32),
            pltpu.VMEM((BATCH,), jnp.int32),
            pltpu.VMEM((IRING, D, 128), jnp.float32),
            pltpu.VMEM((128, 128), jnp.float32),
            pltpu.VMEM((128,), jnp.int32),
        ]
        + [pltpu.SemaphoreType.DMA] * (IRING + 1)
    ),
    compiler_params=pltpu.CompilerParams(needs_layout_passes=False,
                                         disable_bounds_checks=True),
)
def _gather_items(item_hbm, itemT_hbm, icols_hbm,
                  iidx_v, seen_v, ib_v, bb_v, oi_v, ob_v,
                  ring_v, stage_v, bpos_v, *sems):
    isems, ssem = sems[:IRING], sems[IRING]
    wid = _wid()
    lo = wid * IPW
    hi = lo + IPW
    col_lo = lo >> 7

    pltpu.sync_copy(item_hbm, iidx_v)

    lane = lax.iota(jnp.int32, 16)
    zero16 = jnp.zeros((16,), jnp.int32)
    for g in range(NCOLP // 16):
        seen_v[pl.ds(g * 16, 16)] = zero16

    def prefill_bpos():
        t16 = jnp.full((16,), TRASH, jnp.int32)
        for t in range(8):
            bpos_v[pl.ds(t * 16, 16)] = t16

    prefill_bpos()

    def scanb(k, ocnt):
        v = iidx_v[pl.ds(k * 16, 16)]
        m = (v >= lo) & (v < hi)
        c = jnp.where(m, (v >> 7) - col_lo, 0)
        cnt, lastm = plsc.scan_count(c, m)
        n = plsc.load_gather(seen_v, [c], mask=m)
        pos = n + cnt - SCAN_BASE
        posc = jnp.minimum(pos, CAP - 1)
        bvec = k * 16 + lane
        okm = m & (pos < CAP)
        slotv = c * CAP + posc
        plsc.store_scatter(ib_v, [slotv], v, mask=okm)
        plsc.store_scatter(bb_v, [slotv], bvec, mask=okm)
        om = m & (pos >= CAP)
        plsc.store_compressed(oi_v.at[pl.ds(ocnt, 16)], v, mask=om)
        plsc.store_compressed(ob_v.at[pl.ds(ocnt, 16)], bvec, mask=om)
        plsc.store_scatter(seen_v, [c], n + cnt + (1 - SCAN_BASE),
                           mask=m & lastm)
        return ocnt + plsc.all_reduce_population_count(om)[0]

    ocnt = lax.fori_loop(0, NVREG, scanb, 0)

    d16 = [lane + 16 * cb for cb in range(D // 16)]

    def fire_col(slot, ck, go):
        @pl.when(go)
        def _():
            off = pl.multiple_of((col_lo + ck) * 128, 128)
            pltpu.async_copy(itemT_hbm.at[:, pl.ds(off, 128)],
                             ring_v.at[slot], isems[slot])

    def flush():
        pltpu.async_copy(stage_v, icols_hbm.at[bpos_v], ssem).wait()
        prefill_bpos()

    def emit(slot, i_e, b_e, widx):
        l = _splat(i_e & 127)
        for cb in range(D // 16):
            stage_v[widx, pl.ds(cb * 16, 16)] = plsc.load_gather(
                ring_v.at[slot], [d16[cb], l])
        plsc.store_scatter(bpos_v, [_splat(widx)], _splat(b_e),
                           mask=lane == 0)
        widx = widx + 1

        @pl.when(widx == 128)
        def _():
            flush()

        return jnp.where(widx == 128, 0, widx)

    sv0 = seen_v[pl.ds(0, 16)]
    for p in range(ILOOK):
        fire_col(p % IRING, p, sv0[p] > 0)

    def colgroup(g, widx):
        p0 = g * 16
        sv = seen_v[pl.ds(p0, 16)]
        svn = seen_v[pl.ds(jnp.minimum(p0 + 16, NCOLP - 16), 16)]
        for k in range(16):
            pf = p0 + k + ILOOK
            cnt_f = sv[k + ILOOK] if k + ILOOK < 16 else svn[k + ILOOK - 16]
            fire_col((k + ILOOK) % IRING, pf, (pf < NCOLP) & (cnt_f > 0))

            slot = k % IRING
            cntk = sv[k]
            ckv = _splat(p0 + k)

            def body(widx, slot=slot, cntk=cntk, ckv=ckv):
                pltpu.make_async_copy(itemT_hbm.at[:, pl.ds(0, 128)],
                                      ring_v.at[slot], isems[slot]).wait()

                def elem(e, widx):
                    ev = ckv * CAP + _splat(e)
                    i_e = plsc.load_gather(ib_v, [ev])[0]
                    b_e = plsc.load_gather(bb_v, [ev])[0]
                    return emit(slot, i_e, b_e, widx)

                return lax.fori_loop(0, jnp.minimum(cntk, CAP), elem, widx)

            widx = lax.cond(cntk > 0, body, lambda w: w, widx)
        return widx

    widx = lax.fori_loop(0, NCOLP // 16, colgroup, 0)

    def oflow(e, widx):
        i_e = plsc.load_gather(oi_v, [_splat(e)])[0]
        b_e = plsc.load_gather(ob_v, [_splat(e)])[0]
        fire_col(0, (i_e >> 7) - col_lo, True)
        pltpu.make_async_copy(itemT_hbm.at[:, pl.ds(0, 128)],
                              ring_v.at[0], isems[0]).wait()
        return emit(0, i_e, b_e, widx)

    widx = lax.fori_loop(0, ocnt, oflow, widx)

    @pl.when(widx > 0)
    def _():
        flush()


@functools.partial(
    pl.kernel,
    mesh=_mesh,
    out_type=jax.ShapeDtypeStruct((BATCH,), jnp.float32),
    scratch_types=(
        [
            pltpu.VMEM((BPW // 128, 128), jnp.int32),
            pltpu.VMEM((BPW, 128), jnp.float32),
            pltpu.VMEM((2, 128, 128), jnp.float32),
            pltpu.VMEM((BPW,), jnp.float32),
        ]
        + [pltpu.SemaphoreType.DMA] * 3
    ),
    compiler_params=pltpu.CompilerParams(needs_layout_passes=False,
                                         disable_bounds_checks=True),
)
def _pair_dot(user_hbm, upad_hbm, icols_hbm, out_hbm,
              uidx_v, vrows_v, urows_v, out_v, usem0, usem1, vsem):
    usems = (usem0, usem1)
    wid = _wid()
    base = wid * BPW
    nb = BPW // 128

    pltpu.sync_copy(user_hbm.at[pl.ds(wid * nb, nb)], uidx_v)
    vcopy = pltpu.async_copy(icols_hbm.at[pl.ds(base, BPW)], vrows_v, vsem)

    def fire(bi):
        pltpu.async_copy(upad_hbm.at[uidx_v.at[bi]],
                         urows_v.at[bi % 2], usems[bi % 2])

    fire(0)
    fire(1)
    vcopy.wait()

    lane = lax.iota(jnp.int32, 16)

    for bi in range(nb):
        sl = bi % 2
        pltpu.make_async_copy(upad_hbm.at[uidx_v.at[0]],
                              urows_v.at[sl], usems[sl]).wait()

        def blkloop(blk, carry, sl=sl, bi=bi):
            j0 = bi * 128 + blk * 16
            jj0 = blk * 16
            acc = jnp.zeros((16,), jnp.float32)
            for r in range(16):
                j = j0 + r
                jj = jj0 + r
                p = (urows_v[sl, jj, pl.ds(0, 16)]
                     * vrows_v[j, pl.ds(0, 16)])
                for cb in range(1, D // 16):
                    p = p + (urows_v[sl, jj, pl.ds(cb * 16, 16)]
                             * vrows_v[j, pl.ds(cb * 16, 16)])
                acc = jnp.where(lane == r, jnp.sum(p), acc)
            out_v[pl.ds(j0, 16)] = acc
            return carry

        lax.fori_loop(0, 8, blkloop, 0)
        if bi + 2 < nb:
            fire(bi + 2)

    pltpu.sync_copy(out_v, out_hbm.at[pl.ds(base, BPW)])


def kernel(user, item, user_emb, item_emb):
    icols = _gather_items(item.astype(jnp.int32), item_emb.T)
    user2d = user.astype(jnp.int32).reshape(BATCH // 128, 128)
    upad = jnp.pad(user_emb, ((0, 0), (0, 128 - D)))
    return _pair_dot(user2d, upad, icols)


# speedup vs baseline: 1.2408x; 1.0772x over previous
import functools

import jax
import jax.numpy as jnp
from jax import lax
from jax.experimental import pallas as pl
from jax.experimental.pallas import tpu as pltpu
from jax.experimental.pallas import tpu_sc as plsc

BATCH = 16384
D = 64
ITEMS = 1_000_000
NC = 2
NS = 16
NW = NC * NS
BPW = BATCH // NW
IPW = ITEMS // NW
NVREG = BATCH // 16
CAP = 16
NCOLP = 256
IRING = 8
ILOOK = 6
URING = 8
ULOOK = 4
TRASH = BATCH
SROWS = BATCH + 8
SCAN_BASE = 1

_mesh = plsc.VectorSubcoreMesh(core_axis_name="c", subcore_axis_name="s")


def _wid():
    return lax.axis_index("s") * NC + lax.axis_index("c")


def _splat(x):
    return jnp.broadcast_to(x, (16,))


@functools.partial(
    pl.kernel,
    mesh=_mesh,
    out_type=jax.ShapeDtypeStruct((SROWS, 128), jnp.float32),
    scratch_types=(
        [
            pltpu.VMEM((BATCH,), jnp.int32),
            pltpu.VMEM((NCOLP,), jnp.int32),
            pltpu.VMEM((NCOLP * CAP,), jnp.int32),
            pltpu.VMEM((BATCH,), jnp.int32),
            pltpu.VMEM((IRING, D, 128), jnp.float32),
            pltpu.VMEM((128, 128), jnp.float32),
            pltpu.VMEM((128,), jnp.int32),
        ]
        + [pltpu.SemaphoreType.DMA] * (IRING + 1)
    ),
    compiler_params=pltpu.CompilerParams(needs_layout_passes=False,
                                         disable_bounds_checks=True),
)
def _gather_items(item_hbm, itemT_hbm, icols_hbm,
                  iidx_v, seen_v, bb_v, ob_v,
                  ring_v, stage_v, bpos_v, *sems):
    isems, ssem = sems[:IRING], sems[IRING]
    wid = _wid()
    lo = wid * IPW
    hi = lo + IPW
    col_lo = lo >> 7

    pltpu.sync_copy(item_hbm, iidx_v)

    lane = lax.iota(jnp.int32, 16)
    zero16 = jnp.zeros((16,), jnp.int32)
    for g in range(NCOLP // 16):
        seen_v[pl.ds(g * 16, 16)] = zero16

    def prefill_bpos():
        t16 = jnp.full((16,), TRASH, jnp.int32)
        for t in range(8):
            bpos_v[pl.ds(t * 16, 16)] = t16

    prefill_bpos()

    def scanb(k, ocnt):
        v = iidx_v[pl.ds(k * 16, 16)]
        m = (v >= lo) & (v < hi)
        c = jnp.where(m, (v >> 7) - col_lo, 0)
        cnt, lastm = plsc.scan_count(c, m)
        n = plsc.load_gather(seen_v, [c], mask=m)
        pos = n + cnt - SCAN_BASE
        posc = jnp.minimum(pos, CAP - 1)
        bvec = k * 16 + lane
        okm = m & (pos < CAP)
        slotv = c * CAP + posc
        plsc.store_scatter(bb_v, [slotv], bvec, mask=okm)
        om = m & (pos >= CAP)
        plsc.store_compressed(ob_v.at[pl.ds(ocnt, 16)], bvec, mask=om)
        plsc.store_scatter(seen_v, [c], n + cnt + (1 - SCAN_BASE),
                           mask=m & lastm)
        return ocnt + plsc.all_reduce_population_count(om)[0]

    ocnt = lax.fori_loop(0, NVREG, scanb, 0)

    d16 = [lane + 16 * cb for cb in range(D // 16)]

    def fire_col(slot, ck, go):
        @pl.when(go)
        def _():
            off = pl.multiple_of((col_lo + ck) * 128, 128)
            pltpu.async_copy(itemT_hbm.at[:, pl.ds(off, 128)],
                             ring_v.at[slot], isems[slot])

    def flush():
        pltpu.async_copy(stage_v, icols_hbm.at[bpos_v], ssem).wait()
        prefill_bpos()

    def emit(slot, i_e, b_e, widx):
        l = _splat(i_e & 127)
        for cb in range(D // 16):
            stage_v[widx, pl.ds(cb * 16, 16)] = plsc.load_gather(
                ring_v.at[slot], [d16[cb], l])
        plsc.store_scatter(bpos_v, [_splat(widx)], _splat(b_e),
                           mask=lane == 0)
        widx = widx + 1

        @pl.when(widx == 128)
        def _():
            flush()

        return jnp.where(widx == 128, 0, widx)

    sv0 = seen_v[pl.ds(0, 16)]
    for p in range(ILOOK):
        fire_col(p % IRING, p, sv0[p] > 0)

    def colgroup(g, widx):
        p0 = g * 16
        sv = seen_v[pl.ds(p0, 16)]
        svn = seen_v[pl.ds(jnp.minimum(p0 + 16, NCOLP - 16), 16)]
        for k in range(16):
            pf = p0 + k + ILOOK
            cnt_f = sv[k + ILOOK] if k + ILOOK < 16 else svn[k + ILOOK - 16]
            fire_col((k + ILOOK) % IRING, pf, (pf < NCOLP) & (cnt_f > 0))

            slot = k % IRING
            cntk = sv[k]
            ckv = _splat(p0 + k)

            def body(widx, slot=slot, cntk=cntk, ckv=ckv):
                pltpu.make_async_copy(itemT_hbm.at[:, pl.ds(0, 128)],
                                      ring_v.at[slot], isems[slot]).wait()

                def elem(e, widx):
                    ev = ckv * CAP + _splat(e)
                    b_e = plsc.load_gather(bb_v, [ev])[0]
                    i_e = plsc.load_gather(iidx_v, [_splat(b_e)])[0]
                    return emit(slot, i_e, b_e, widx)

                return lax.fori_loop(0, jnp.minimum(cntk, CAP), elem, widx)

            widx = lax.cond(cntk > 0, body, lambda w: w, widx)
        return widx

    widx = lax.fori_loop(0, NCOLP // 16, colgroup, 0)

    def oflow(e, widx):
        b_e = plsc.load_gather(ob_v, [_splat(e)])[0]
        i_e = plsc.load_gather(iidx_v, [_splat(b_e)])[0]
        fire_col(0, (i_e >> 7) - col_lo, True)
        pltpu.make_async_copy(itemT_hbm.at[:, pl.ds(0, 128)],
                              ring_v.at[0], isems[0]).wait()
        return emit(0, i_e, b_e, widx)

    widx = lax.fori_loop(0, ocnt, oflow, widx)

    @pl.when(widx > 0)
    def _():
        flush()


@functools.partial(
    pl.kernel,
    mesh=_mesh,
    out_type=jax.ShapeDtypeStruct((BATCH,), jnp.float32),
    scratch_types=(
        [
            pltpu.VMEM((BPW // 128, 128), jnp.int32),
            pltpu.VMEM((BPW, 128), jnp.float32),
            pltpu.VMEM((2, 128, 128), jnp.float32),
            pltpu.VMEM((BPW,), jnp.float32),
        ]
        + [pltpu.SemaphoreType.DMA] * 3
    ),
    compiler_params=pltpu.CompilerParams(needs_layout_passes=False,
                                         disable_bounds_checks=True),
)
def _pair_dot(user_hbm, upad_hbm, icols_hbm, out_hbm,
              uidx_v, vrows_v, urows_v, out_v, usem0, usem1, vsem):
    usems = (usem0, usem1)
    wid = _wid()
    base = wid * BPW
    nb = BPW // 128

    pltpu.sync_copy(user_hbm.at[pl.ds(wid * nb, nb)], uidx_v)
    vcopy = pltpu.async_copy(icols_hbm.at[pl.ds(base, BPW)], vrows_v, vsem)

    def fire(bi):
        pltpu.async_copy(upad_hbm.at[uidx_v.at[bi]],
                         urows_v.at[bi % 2], usems[bi % 2])

    fire(0)
    fire(1)
    vcopy.wait()

    lane = lax.iota(jnp.int32, 16)

    for bi in range(nb):
        sl = bi % 2
        pltpu.make_async_copy(upad_hbm.at[uidx_v.at[0]],
                              urows_v.at[sl], usems[sl]).wait()

        def blkloop(blk, carry, sl=sl, bi=bi):
            j0 = bi * 128 + blk * 16
            jj0 = blk * 16
            acc = jnp.zeros((16,), jnp.float32)
            for r in range(16):
                j = j0 + r
                jj = jj0 + r
                p = (urows_v[sl, jj, pl.ds(0, 16)]
                     * vrows_v[j, pl.ds(0, 16)])
                for cb in range(1, D // 16):
                    p = p + (urows_v[sl, jj, pl.ds(cb * 16, 16)]
                             * vrows_v[j, pl.ds(cb * 16, 16)])
                acc = jnp.where(lane == r, jnp.sum(p), acc)
            out_v[pl.ds(j0, 16)] = acc
            return carry

        lax.fori_loop(0, 8, blkloop, 0)
        if bi + 2 < nb:
            fire(bi + 2)

    pltpu.sync_copy(out_v, out_hbm.at[pl.ds(base, BPW)])


def kernel(user, item, user_emb, item_emb):
    icols = _gather_items(item.astype(jnp.int32), item_emb.T)
    user2d = user.astype(jnp.int32).reshape(BATCH // 128, 128)
    upad = jnp.pad(user_emb, ((0, 0), (0, 128 - D)))
    return _pair_dot(user2d, upad, icols)


# speedup vs baseline: 1.2460x; 1.0042x over previous
import functools

import jax
import jax.numpy as jnp
from jax import lax
from jax.experimental import pallas as pl
from jax.experimental.pallas import tpu as pltpu
from jax.experimental.pallas import tpu_sc as plsc

BATCH = 16384
D = 64
NC = 2
NS = 16
NW = NC * NS
BPW = BATCH // NW
NBLK = BPW // 16
RING = 8
LOOK = 4

_mesh = plsc.VectorSubcoreMesh(core_axis_name="c", subcore_axis_name="s")


@functools.partial(
    pl.kernel,
    mesh=_mesh,
    out_type=jax.ShapeDtypeStruct((BATCH,), jnp.float32),
    scratch_types=(
        [
            pltpu.VMEM((BPW,), jnp.int32),
            pltpu.VMEM((BPW,), jnp.int32),
            pltpu.VMEM((RING, 8, D), jnp.float32),
            pltpu.VMEM((RING, D, 128), jnp.float32),
            pltpu.VMEM((BPW,), jnp.float32),
        ]
        + [pltpu.SemaphoreType.DMA] * (2 * RING)
    ),
    compiler_params=pltpu.CompilerParams(needs_layout_passes=False,
                                         disable_bounds_checks=True),
)
def _bpr_sc(user_hbm, item_hbm, uemb_hbm, itemT_hbm, out_hbm,
            uidx_v, iidx_v, uring_v, iring_v, out_v, *sems):
    usems, isems = sems[:RING], sems[RING:]
    wid = lax.axis_index("s") * NC + lax.axis_index("c")
    base = wid * BPW

    pltpu.sync_copy(user_hbm.at[pl.ds(base, BPW)], uidx_v)
    pltpu.sync_copy(item_hbm.at[pl.ds(base, BPW)], iidx_v)

    def fire(slot, uv, iv):
        uoff = pl.multiple_of((uv >> 3) * 8, 8)
        pltpu.async_copy(uemb_hbm.at[pl.ds(uoff, 8), :],
                         uring_v.at[slot], usems[slot])
        ioff = pl.multiple_of((iv >> 7) * 128, 128)
        pltpu.async_copy(itemT_hbm.at[:, pl.ds(ioff, 128)],
                         iring_v.at[slot], isems[slot])

    uvec0 = uidx_v[pl.ds(0, 16)]
    ivec0 = iidx_v[pl.ds(0, 16)]
    for j in range(LOOK):
        fire(j % RING, uvec0[j], ivec0[j])

    lane = lax.iota(jnp.int32, 16)
    d16 = [lane + 16 * cb for cb in range(D // 16)]

    def block(blk, carry):
        j0 = blk * 16
        uvec = uidx_v[pl.ds(j0, 16)]
        ivec = iidx_v[pl.ds(j0, 16)]
        nxt0 = jnp.minimum(j0 + 16, BPW - 16)
        uvec_n = uidx_v[pl.ds(nxt0, 16)]
        ivec_n = iidx_v[pl.ds(nxt0, 16)]
        acc = jnp.zeros((16,), jnp.float32)
        for r in range(16):
            j = j0 + r
            if r + LOOK < 16:
                uvf, ivf = uvec[r + LOOK], ivec[r + LOOK]
            else:
                uvf, ivf = uvec_n[r + LOOK - 16], ivec_n[r + LOOK - 16]
            slot_f = (r + LOOK) % RING

            @pl.when(j + LOOK < BPW)
            def _():
                fire(slot_f, uvf, ivf)

            slot = r % RING
            pltpu.make_async_copy(uemb_hbm.at[pl.ds(0, 8), :],
                                  uring_v.at[slot], usems[slot]).wait()
            pltpu.make_async_copy(itemT_hbm.at[:, pl.ds(0, 128)],
                                  iring_v.at[slot], isems[slot]).wait()
            urow = uvec[r] & 7
            l = jnp.broadcast_to(ivec[r] & 127, (16,))
            p = (uring_v[slot, urow, pl.ds(0, 16)]
                 * plsc.load_gather(iring_v.at[slot], [d16[0], l]))
            for cb in range(1, D // 16):
                p = p + (uring_v[slot, urow, pl.ds(cb * 16, 16)]
                         * plsc.load_gather(iring_v.at[slot], [d16[cb], l]))
            acc = jnp.where(lane == r, jnp.sum(p), acc)
        out_v[pl.ds(j0, 16)] = acc
        return carry

    lax.fori_loop(0, NBLK, block, 0)
    pltpu.sync_copy(out_v, out_hbm.at[pl.ds(base, BPW)])


def kernel(user, item, user_emb, item_emb):
    return _bpr_sc(user.astype(jnp.int32), item.astype(jnp.int32),
                   user_emb, item_emb.T)
